# baseline (speedup 1.0000x reference)
import functools
import jax
import jax.numpy as jnp
from jax import lax
from jax.experimental import pallas as pl
from jax.experimental.pallas import tpu as pltpu
from jax.experimental.pallas import tpu_sc as plsc

VOCAB = 1000000
EMBED = 64
HIDDEN = 128
CLASSES = 3
B = 4096
L = 200

NC = 2
NS = 16
NW = NC * NS
BPW = B // NW
CHUNK = 100
CPR = L // CHUNK
NBUF = 4


def _sc_pool_body(table_hbm, idx_hbm, out_hbm, idx_v, b0, b1v, b2v, b3, pooled_v,
                  s0, s1, s2, s3):
    bufs = (b0, b1v, b2v, b3)
    sems = (s0, s1, s2, s3)
    wid = lax.axis_index("s") * NC + lax.axis_index("c")
    row_base = wid * BPW
    chunk_base = wid * (BPW * CPR)

    pltpu.sync_copy(idx_hbm.at[pl.ds(chunk_base, BPW * CPR)], idx_v)

    def issue(row, slot):
        pltpu.async_copy(table_hbm.at[idx_v.at[CPR * row]],
                         bufs[slot].at[pl.ds(0, CHUNK)], sems[slot])
        pltpu.async_copy(table_hbm.at[idx_v.at[CPR * row + 1]],
                         bufs[slot].at[pl.ds(CHUNK, CHUNK)], sems[slot])

    for s in range(NBUF):
        issue(s, s)

    def accum_body(j, acc, buf):
        return tuple(acc[k] + buf[j, pl.ds(16 * k, 16)] for k in range(4))

    def outer(g, carry):
        for s in range(NBUF):
            r = g * NBUF + s
            pltpu.make_async_copy(table_hbm.at[idx_v.at[CPR * r]],
                                  bufs[s].at[pl.ds(0, CHUNK)], sems[s]).wait()
            pltpu.make_async_copy(table_hbm.at[idx_v.at[CPR * r + 1]],
                                  bufs[s].at[pl.ds(CHUNK, CHUNK)], sems[s]).wait()
            z = jnp.zeros((16,), jnp.float32)
            acc = lax.fori_loop(0, L,
                                functools.partial(accum_body, buf=bufs[s]),
                                (z, z, z, z), unroll=4)
            for k in range(4):
                pooled_v[r, pl.ds(16 * k, 16)] = acc[k]

            @pl.when(r + NBUF < BPW)
            def _():
                issue(r + NBUF, s)
        return carry

    lax.fori_loop(0, BPW // NBUF, outer, 0)
    pltpu.sync_copy(pooled_v, out_hbm.at[pl.ds(row_base, BPW)])


def _sc_pool(table, idx_chunks):
    mesh = plsc.VectorSubcoreMesh(core_axis_name="c", subcore_axis_name="s",
                                  num_cores=NC, num_subcores=NS)
    f = pl.kernel(
        _sc_pool_body,
        out_type=jax.ShapeDtypeStruct((B, EMBED), jnp.float32),
        mesh=mesh,
        scratch_types=[
            pltpu.VMEM((BPW * CPR, CHUNK), jnp.int32),
            pltpu.VMEM((L, EMBED), jnp.float32),
            pltpu.VMEM((L, EMBED), jnp.float32),
            pltpu.VMEM((L, EMBED), jnp.float32),
            pltpu.VMEM((L, EMBED), jnp.float32),
            pltpu.VMEM((BPW, EMBED), jnp.float32),
            pltpu.SemaphoreType.DMA,
            pltpu.SemaphoreType.DMA,
            pltpu.SemaphoreType.DMA,
            pltpu.SemaphoreType.DMA,
        ],
        compiler_params=pltpu.CompilerParams(use_tc_tiling_on_sc=False),
    )
    return f(table, idx_chunks)


def _mlp_body(p_ref, w1t_ref, b1_ref, w2t_ref, b2_ref, o_ref):
    p = p_ref[...] * (1.0 / L)
    h = jnp.dot(p, w1t_ref[...], preferred_element_type=jnp.float32)
    h = jnp.maximum(h + b1_ref[...], 0.0)
    o_ref[...] = jnp.dot(h, w2t_ref[...],
                         preferred_element_type=jnp.float32) + b2_ref[...]


def _mlp(pooled_sum, w1t, b1, w2t_pad, b2_pad):
    blk = 512
    grid = (B // blk,)
    return pl.pallas_call(
        _mlp_body,
        grid=grid,
        in_specs=[
            pl.BlockSpec((blk, EMBED), lambda i: (i, 0)),
            pl.BlockSpec((EMBED, HIDDEN), lambda i: (0, 0)),
            pl.BlockSpec((1, HIDDEN), lambda i: (0, 0)),
            pl.BlockSpec((HIDDEN, HIDDEN), lambda i: (0, 0)),
            pl.BlockSpec((1, HIDDEN), lambda i: (0, 0)),
        ],
        out_specs=pl.BlockSpec((blk, HIDDEN), lambda i: (i, 0)),
        out_shape=jax.ShapeDtypeStruct((B, HIDDEN), jnp.float32),
    )(pooled_sum, w1t, b1, w2t_pad, b2_pad)


def kernel(x, table, W1, b1, W2, b2):
    idx_chunks = x.astype(jnp.int32).reshape(B * CPR, CHUNK)
    pooled_sum = _sc_pool(table, idx_chunks)
    w1t = W1.T
    b1r = b1.reshape(1, HIDDEN)
    w2t_pad = jnp.zeros((HIDDEN, HIDDEN), jnp.float32).at[:, :CLASSES].set(W2.T)
    b2_pad = jnp.zeros((1, HIDDEN), jnp.float32).at[0, :CLASSES].set(b2)
    out_pad = _mlp(pooled_sum, w1t, b1r, w2t_pad, b2_pad)
    return out_pad[:, :CLASSES]
